# baseline (device time: 16325 ns/iter reference)
import jax
import jax.numpy as jnp
from jax import lax
from jax.experimental import pallas as pl
from jax.experimental.pallas import tpu as pltpu

N_DEV = 4
HALF = 128
CHUNKS = 8
CH = HALF // CHUNKS
N_EX = 3 * 2 * CHUNKS


def kernel(x):
    _, m, n = x.shape

    def body(x_ref, out_ref, recv_ref, send_sems, recv_sems):
        me = lax.axis_index("i")
        p1 = me ^ 1
        p2 = 3 - me

        h_a = (me ^ (me // 2)) % 2
        h_b = me // 2

        keep_off = [h_a * HALF, 2 * HALF + h_b * HALF]
        send_off = [(1 - h_a) * HALF, 2 * HALF + (1 - h_b) * HALF]
        partner = [[p1, p2, p1], [p2, p1, p2]]

        barrier_sem = pltpu.get_barrier_semaphore()
        for nbr in [p1, p2]:
            pl.semaphore_signal(
                barrier_sem, inc=1,
                device_id=(nbr,), device_id_type=pl.DeviceIdType.MESH,
            )
        pl.semaphore_wait(barrier_sem, 2)

        def slot(rnd, blk, c):
            return (rnd * 2 + blk) * CHUNKS + c

        def exchange(sl, src, part):
            rdma = pltpu.make_async_remote_copy(
                src_ref=src,
                dst_ref=recv_ref.at[sl],
                send_sem=send_sems.at[sl],
                recv_sem=recv_sems.at[sl],
                device_id=(part,),
                device_id_type=pl.DeviceIdType.MESH,
            )
            rdma.start()
            return rdma

        inflight = {}
        for c in range(CHUNKS):
            for blk in (0, 1):
                src = x_ref.at[0, pl.ds(send_off[blk] + c * CH, CH), :]
                inflight[(0, blk, c)] = exchange(
                    slot(0, blk, c), src, partner[blk][0]
                )

        for rnd in range(3):
            for c in range(CHUNKS):
                for blk in (0, 1):
                    sl = slot(rnd, blk, c)
                    ko = keep_off[blk] + c * CH
                    inflight[(rnd, blk, c)].wait()
                    if rnd == 0:
                        out_ref[pl.ds(ko, CH), :] = (
                            x_ref[0, pl.ds(ko, CH), :] + recv_ref[sl, :, :]
                        )
                        inflight[(1, blk, c)] = exchange(
                            slot(1, blk, c), out_ref.at[pl.ds(ko, CH), :],
                            partner[blk][1],
                        )
                    elif rnd == 1:
                        out_ref[pl.ds(ko, CH), :] += recv_ref[sl, :, :]
                        inflight[(2, blk, c)] = exchange(
                            slot(2, blk, c), out_ref.at[pl.ds(ko, CH), :],
                            partner[blk][2],
                        )
                    else:
                        out_ref[pl.ds(send_off[blk] + c * CH, CH), :] = (
                            recv_ref[sl, :, :]
                        )

    return pl.pallas_call(
        body,
        out_shape=jax.ShapeDtypeStruct((m, n), x.dtype),
        in_specs=[pl.BlockSpec(memory_space=pltpu.VMEM)],
        out_specs=pl.BlockSpec(memory_space=pltpu.VMEM),
        scratch_shapes=[
            pltpu.VMEM((N_EX, CH, n), x.dtype),
            pltpu.SemaphoreType.DMA((N_EX,)),
            pltpu.SemaphoreType.DMA((N_EX,)),
        ],
        compiler_params=pltpu.CompilerParams(collective_id=0),
    )(x)


# device time: 14794 ns/iter; 1.1035x vs baseline; 1.1035x over previous
import jax
import jax.numpy as jnp
from jax import lax
from jax.experimental import pallas as pl
from jax.experimental.pallas import tpu as pltpu

N_DEV = 4
HALF = 128
CHUNKS = 4
CH = HALF // CHUNKS
N_EX = 3 * 2 * CHUNKS


def kernel(x):
    _, m, n = x.shape

    def body(x_ref, out_ref, recv_ref, send_sems, recv_sems):
        me = lax.axis_index("i")
        p1 = me ^ 1
        p2 = 3 - me

        h_a = (me ^ (me // 2)) % 2
        h_b = me // 2

        keep_off = [h_a * HALF, 2 * HALF + h_b * HALF]
        send_off = [(1 - h_a) * HALF, 2 * HALF + (1 - h_b) * HALF]
        partner = [[p1, p2, p1], [p2, p1, p2]]

        barrier_sem = pltpu.get_barrier_semaphore()
        for nbr in [p1, p2]:
            pl.semaphore_signal(
                barrier_sem, inc=1,
                device_id=(nbr,), device_id_type=pl.DeviceIdType.MESH,
            )
        pl.semaphore_wait(barrier_sem, 2)

        def slot(rnd, blk, c):
            return (rnd * 2 + blk) * CHUNKS + c

        def exchange(sl, src, part):
            rdma = pltpu.make_async_remote_copy(
                src_ref=src,
                dst_ref=recv_ref.at[sl],
                send_sem=send_sems.at[sl],
                recv_sem=recv_sems.at[sl],
                device_id=(part,),
                device_id_type=pl.DeviceIdType.MESH,
            )
            rdma.start()
            return rdma

        inflight = {}
        for c in range(CHUNKS):
            for blk in (0, 1):
                src = x_ref.at[0, pl.ds(send_off[blk] + c * CH, CH), :]
                inflight[(0, blk, c)] = exchange(
                    slot(0, blk, c), src, partner[blk][0]
                )

        for rnd in range(3):
            for c in range(CHUNKS):
                for blk in (0, 1):
                    sl = slot(rnd, blk, c)
                    ko = keep_off[blk] + c * CH
                    inflight[(rnd, blk, c)].wait()
                    if rnd == 0:
                        out_ref[pl.ds(ko, CH), :] = (
                            x_ref[0, pl.ds(ko, CH), :] + recv_ref[sl, :, :]
                        )
                        inflight[(1, blk, c)] = exchange(
                            slot(1, blk, c), out_ref.at[pl.ds(ko, CH), :],
                            partner[blk][1],
                        )
                    elif rnd == 1:
                        out_ref[pl.ds(ko, CH), :] += recv_ref[sl, :, :]
                        inflight[(2, blk, c)] = exchange(
                            slot(2, blk, c), out_ref.at[pl.ds(ko, CH), :],
                            partner[blk][2],
                        )
                    else:
                        out_ref[pl.ds(send_off[blk] + c * CH, CH), :] = (
                            recv_ref[sl, :, :]
                        )

    return pl.pallas_call(
        body,
        out_shape=jax.ShapeDtypeStruct((m, n), x.dtype),
        in_specs=[pl.BlockSpec(memory_space=pltpu.VMEM)],
        out_specs=pl.BlockSpec(memory_space=pltpu.VMEM),
        scratch_shapes=[
            pltpu.VMEM((N_EX, CH, n), x.dtype),
            pltpu.SemaphoreType.DMA((N_EX,)),
            pltpu.SemaphoreType.DMA((N_EX,)),
        ],
        compiler_params=pltpu.CompilerParams(collective_id=0),
    )(x)
